# Initial kernel scaffold; baseline (speedup 1.0000x reference)
import jax
import jax.numpy as jnp
from jax.experimental import pallas as pl


def kernel(y_s, y_s_adv, labels_s, y_t, y_t_adv, epoch):
    raise NotImplementedError("write your pallas kernel here")


# speedup vs baseline: 4.5938x; 4.5938x over previous
import functools

import jax
import jax.numpy as jnp
from jax.experimental import pallas as pl

_C = 10
_B = 2048
_EPS = 0.05
_ROWS = 256


def _softmax(x):
    m = jnp.max(x, axis=1, keepdims=True)
    e = jnp.exp(x - m)
    return e / jnp.sum(e, axis=1, keepdims=True)


def _pair_loss(x, eps):
    return jnp.log(1.0 + jnp.exp(-(x - eps))) + jnp.log(1.0 + jnp.exp(x + eps))


def _auc_kernel(ys_ref, ysa_ref, ysb_ref, ysab_ref, labc_ref, labr_ref,
                emp_ref, disc_ref):
    i = pl.program_id(0)

    ys = ys_ref[...]
    ysa = ysa_ref[...]
    lab_col = labc_ref[...]
    lab_row = labr_ref[...]

    p_full = _softmax(ys)
    pa_full = _softmax(ysa)

    cls = jax.lax.broadcasted_iota(jnp.int32, (1, _C), 1)
    onehot = (lab_col == cls).astype(jnp.float32)

    dot = functools.partial(
        jax.lax.dot_general,
        dimension_numbers=(((1,), (1,)), ((), ())),
        preferred_element_type=jnp.float32,
        precision=jax.lax.Precision.HIGHEST,
    )
    m = dot(onehot, p_full)
    ma = dot(onehot, pa_full)

    p_blk = _softmax(ysb_ref[...])
    pa_blk = _softmax(ysab_ref[...])
    g = jnp.sum(onehot * p_blk, axis=1, keepdims=True)
    ga = jnp.sum(onehot * pa_blk, axis=1, keepdims=True)

    w = jnp.zeros_like(g)
    labr_f = lab_row
    for c in range(_C):
        n_c = jnp.sum((labr_f == c).astype(jnp.float32))
        fac_c = 1.0 / (n_c * (_B - n_c))
        w = w + jnp.where(lab_col == c, fac_c, 0.0)

    valid = (lab_col != lab_row).astype(jnp.float32)
    wv = w * valid

    e_x = 4.0 * (1.0 - g + m)
    s_x = 2.0 * ((ga - g) - ma + m)
    emp = jnp.sum(wv * _pair_loss(e_x, _EPS)).reshape(1, 1)
    disc = jnp.sum(wv * _pair_loss(s_x, _EPS)).reshape(1, 1)

    @pl.when(i == 0)
    def _init():
        emp_ref[...] = jnp.zeros((1, 1), jnp.float32)
        disc_ref[...] = jnp.zeros((1, 1), jnp.float32)

    emp_ref[...] += emp
    disc_ref[...] += disc


def kernel(y_s, y_s_adv, labels_s, y_t, y_t_adv, epoch):
    lab = labels_s.astype(jnp.int32)
    lab_col = lab.reshape(_B, 1)
    lab_row = lab.reshape(1, _B)

    grid = (_B // _ROWS,)
    emp, disc = pl.pallas_call(
        _auc_kernel,
        grid=grid,
        in_specs=[
            pl.BlockSpec((_B, _C), lambda i: (0, 0)),
            pl.BlockSpec((_B, _C), lambda i: (0, 0)),
            pl.BlockSpec((_ROWS, _C), lambda i: (i, 0)),
            pl.BlockSpec((_ROWS, _C), lambda i: (i, 0)),
            pl.BlockSpec((_ROWS, 1), lambda i: (i, 0)),
            pl.BlockSpec((1, _B), lambda i: (0, 0)),
        ],
        out_specs=[
            pl.BlockSpec((1, 1), lambda i: (0, 0)),
            pl.BlockSpec((1, 1), lambda i: (0, 0)),
        ],
        out_shape=[
            jax.ShapeDtypeStruct((1, 1), jnp.float32),
            jax.ShapeDtypeStruct((1, 1), jnp.float32),
        ],
    )(y_s, y_s_adv, y_s, y_s_adv, lab_col, lab_row)

    empirical = 0.25 * emp[0, 0]
    transfer = -0.5 * disc[0, 0]
    return (empirical, transfer)


# speedup vs baseline: 5.0662x; 1.1028x over previous
import functools
import math

import jax
import jax.numpy as jnp
from jax.experimental import pallas as pl

_C = 10
_B = 2048
_EPS = 0.05
_ROWS = 256
_K0 = 1.0 + math.exp(2.0 * _EPS)
_K2 = math.exp(2.0 * _EPS)


def _softmax(x):
    m = jnp.max(x, axis=1, keepdims=True)
    e = jnp.exp(x - m)
    return e / jnp.sum(e, axis=1, keepdims=True)


def _auc_kernel(ys_ref, ysa_ref, ysb_ref, ysab_ref, labc_ref, labr_ref,
                emp_ref, disc_ref):
    i = pl.program_id(0)

    lab_col = labc_ref[...]
    lab_row = labr_ref[...]

    p_full = _softmax(ys_ref[...])
    pa_full = _softmax(ysa_ref[...])
    e2p = jnp.exp(2.0 * p_full)
    e2pai = jnp.exp(-2.0 * pa_full)

    cls = jax.lax.broadcasted_iota(jnp.int32, (1, _C), 1)
    onehot = (lab_col == cls).astype(jnp.float32)

    dot = functools.partial(
        jax.lax.dot_general,
        dimension_numbers=(((1,), (1,)), ((), ())),
        preferred_element_type=jnp.float32,
        precision=jax.lax.Precision.HIGHEST,
    )
    u2 = dot(onehot, e2p)
    v = dot(onehot, e2pai)

    p_blk = _softmax(ysb_ref[...])
    pa_blk = _softmax(ysab_ref[...])
    g = jnp.sum(onehot * p_blk, axis=1, keepdims=True)
    ga = jnp.sum(onehot * pa_blk, axis=1, keepdims=True)
    c_e = jnp.exp(_EPS + 4.0 * (1.0 - g))
    c_s = jnp.exp(_EPS + 2.0 * (ga - g))

    w = jnp.zeros_like(g)
    for c in range(_C):
        n_c = jnp.sum((lab_row == c).astype(jnp.float32))
        fac_c = math.log(2.0) / (n_c * (_B - n_c))
        w = w + jnp.where(lab_col == c, fac_c, 0.0)
    wv = jnp.where(lab_col != lab_row, w, 0.0)

    h_e = (c_e * u2) * u2
    h_s = (c_s * u2) * v
    l_e = jnp.log2(_K0 + h_e + _K2 / h_e)
    l_s = jnp.log2(_K0 + h_s + _K2 / h_s)
    emp = jnp.sum(wv * l_e).reshape(1, 1)
    disc = jnp.sum(wv * l_s).reshape(1, 1)

    @pl.when(i == 0)
    def _init():
        emp_ref[...] = jnp.zeros((1, 1), jnp.float32)
        disc_ref[...] = jnp.zeros((1, 1), jnp.float32)

    emp_ref[...] += emp
    disc_ref[...] += disc


def kernel(y_s, y_s_adv, labels_s, y_t, y_t_adv, epoch):
    lab = labels_s.astype(jnp.int32)
    lab_col = lab.reshape(_B, 1)
    lab_row = lab.reshape(1, _B)

    grid = (_B // _ROWS,)
    emp, disc = pl.pallas_call(
        _auc_kernel,
        grid=grid,
        in_specs=[
            pl.BlockSpec((_B, _C), lambda i: (0, 0)),
            pl.BlockSpec((_B, _C), lambda i: (0, 0)),
            pl.BlockSpec((_ROWS, _C), lambda i: (i, 0)),
            pl.BlockSpec((_ROWS, _C), lambda i: (i, 0)),
            pl.BlockSpec((_ROWS, 1), lambda i: (i, 0)),
            pl.BlockSpec((1, _B), lambda i: (0, 0)),
        ],
        out_specs=[
            pl.BlockSpec((1, 1), lambda i: (0, 0)),
            pl.BlockSpec((1, 1), lambda i: (0, 0)),
        ],
        out_shape=[
            jax.ShapeDtypeStruct((1, 1), jnp.float32),
            jax.ShapeDtypeStruct((1, 1), jnp.float32),
        ],
    )(y_s, y_s_adv, y_s, y_s_adv, lab_col, lab_row)

    empirical = 0.25 * emp[0, 0]
    transfer = -0.5 * disc[0, 0]
    return (empirical, transfer)
